# baseline (speedup 1.0000x reference)
import functools

import jax
import jax.numpy as jnp
from jax import lax
from jax.experimental import pallas as pl
from jax.experimental.pallas import tpu as pltpu
from jax.experimental.pallas import tpu_sc as plsc

N = 10000
E = 320000
K = 27
CH = 32
HID = 24
OUT = 8

NPAD = 10240
NB = 128
GRID = NPAD // NB
NSUB = 16
ECH = 96
NCHUNK = 216
IB = 24
NIB = NCHUNK // IB
EPAD = NSUB * NCHUNK * ECH
ZROWS = 16
RPS = NPAD // NSUB
FH = 128

f32 = jnp.float32


def _sc_body(y_hbm, gidx_hbm, dst_hbm, out_hbm,
             gidx_v, dst_v, zbuf, gb0, gb1, gb2, acc,
             sg0, sg1, sg2, ss0, ss1, ss2):
    c = lax.axis_index("c")
    s = lax.axis_index("s")
    gb = (gb0, gb1, gb2)
    sg = (sg0, sg1, sg2)
    ss = (ss0, ss1, ss2)

    def zrow(i, carry):
        for j in range(FH // 16):
            zbuf[i, pl.ds(j * 16, 16)] = jnp.zeros((16,), f32)
        return carry
    lax.fori_loop(0, ZROWS, zrow, 0)
    for r in range(RPS // ZROWS):
        pltpu.sync_copy(zbuf, acc.at[pl.ds(s * RPS + r * ZROWS, ZROWS)])
    plsc.subcore_barrier()

    yc = y_hbm.at[c]

    def blk(ib, carry):
        pltpu.sync_copy(gidx_hbm.at[s, pl.ds(ib * IB, IB)], gidx_v.at[0])
        pltpu.sync_copy(dst_hbm.at[s, pl.ds(ib * IB, IB)], dst_v.at[0])

        g_desc = [None, None, None]
        s_desc = [None, None, None]
        g_desc[0] = pltpu.async_copy(yc.at[gidx_v.at[0, 0]], gb[0], sg[0])
        g_desc[1] = pltpu.async_copy(yc.at[gidx_v.at[0, 1]], gb[1], sg[1])
        for j in range(IB):
            b = j % 3
            bp = (j + 2) % 3
            if j + 2 < IB:
                if j >= 1:
                    s_desc[bp].wait()
                g_desc[bp] = pltpu.async_copy(
                    yc.at[gidx_v.at[0, j + 2]], gb[bp], sg[bp])
            g_desc[b].wait()
            s_desc[b] = pltpu.async_copy(
                gb[b], acc.at[dst_v.at[0, j]], ss[b], add=True)
        s_desc[0].wait()
        s_desc[1].wait()
        s_desc[2].wait()
        return carry
    lax.fori_loop(0, NIB, blk, 0)
    plsc.subcore_barrier()

    pltpu.sync_copy(acc.at[pl.ds(s * RPS, RPS)],
                    out_hbm.at[c, pl.ds(s * RPS, RPS)])


def _sc_agg(y, gidx, dst):
    mesh = plsc.VectorSubcoreMesh(core_axis_name="c", subcore_axis_name="s")
    fn = pl.kernel(
        _sc_body,
        out_type=jax.ShapeDtypeStruct((2, NPAD, FH), f32),
        mesh=mesh,
        scratch_types=[
            pltpu.VMEM((1, IB, ECH), jnp.int32),
            pltpu.VMEM((1, IB, ECH), jnp.int32),
            pltpu.VMEM((ZROWS, FH), f32),
            pltpu.VMEM((ECH, FH), f32),
            pltpu.VMEM((ECH, FH), f32),
            pltpu.VMEM((ECH, FH), f32),
            pltpu.VMEM_SHARED((NPAD, FH), f32),
            pltpu.SemaphoreType.DMA,
            pltpu.SemaphoreType.DMA,
            pltpu.SemaphoreType.DMA,
            pltpu.SemaphoreType.DMA,
            pltpu.SemaphoreType.DMA,
            pltpu.SemaphoreType.DMA,
        ],
    )
    return fn(y.reshape(2, K * NPAD, FH), gidx, dst)


def _yspec():
    return pl.BlockSpec((2, K, NB, FH), lambda i: (0, 0, i, 0))


def _aspec():
    return pl.BlockSpec((2, NB, FH), lambda i: (0, i, 0))


def _nspec(w):
    return pl.BlockSpec((NB, w), lambda i: (i, 0))


def _fspec(shape):
    return pl.BlockSpec(shape, lambda i: (0,) * len(shape))


def _cat(a_ref):
    return jnp.concatenate([a_ref[0], a_ref[1]], axis=-1)


def _emit(h8, w_ref, y_ref):
    for k in range(K):
        row = jnp.dot(h8, w_ref[k], preferred_element_type=f32)
        y_ref[0, k] = row[:, :FH]
        y_ref[1, k] = row[:, FH:]


def _t_first(x_ref, w_ref, y_ref):
    x = x_ref[...]
    h8 = jnp.concatenate([x] * OUT, axis=1)
    _emit(h8, w_ref, y_ref)


def _t_lin(a_ref, b0_ref, w_ref, y_ref):
    a = _cat(a_ref)
    h8 = jnp.maximum(a + b0_ref[0], 0.0)
    _emit(h8, w_ref, y_ref)


def _t_res(a1_ref, a3_ref, b0_ref, b2_ref, w_ref, y_ref):
    h = jnp.maximum(_cat(a1_ref) + b0_ref[0], 0.0)
    r2 = _cat(a3_ref) + b2_ref[0]
    h8 = jnp.maximum(h + r2, 0.0)
    _emit(h8, w_ref, y_ref)


def _t_pr(a4_ref, b3_ref, w_ref, y_ref):
    a = _cat(a_ref=a4_ref)
    g = a + b3_ref[0]
    xg = g[:, :CH]
    parts = [xg] + [xg + g[:, oi * CH:(oi + 1) * CH] for oi in range(1, OUT)]
    h8 = jnp.concatenate(parts, axis=1)
    _emit(h8, w_ref, y_ref)


def _t_head(apr_ref, prb_ref, w1_ref, b1_ref, w2_ref, b2_ref, out_ref):
    a = _cat(apr_ref).reshape(NB, OUT, CH)
    cols = []
    for oi in range(OUT):
        t = a[:, oi] + prb_ref[oi]
        y1 = jnp.maximum(
            jnp.dot(t, w1_ref[oi], preferred_element_type=f32) + b1_ref[oi],
            0.0)
        z = jnp.dot(y1, w2_ref[oi], preferred_element_type=f32) + b2_ref[oi]
        cols.append(1.0 / (1.0 + jnp.exp(-z)))
    out_ref[...] = jnp.concatenate(cols, axis=1)


def _call(body, in_arrs, in_specs, out_shape, out_spec):
    return pl.pallas_call(
        body, grid=(GRID,), in_specs=in_specs,
        out_specs=out_spec,
        out_shape=jax.ShapeDtypeStruct(out_shape, f32),
    )(*in_arrs)


def kernel(x_low, x_occ, edge_index, edge_offset,
           bi_W0, bi_b0, bi_W1, bi_b1, bi_W2, bi_b2, bi_W3, bi_b3,
           pr_W, pr_b, ml_W1, ml_b1, ml_W2, ml_b2,
           ot_W0, ot_b0, ot_W1, ot_b1, ot_W2, ot_b2, ot_W3, ot_b3):
    src = edge_index[0]
    dstv = edge_index[1]
    off = edge_offset

    gidx = off * NPAD + src
    pad = EPAD - E
    gidx_t = jnp.concatenate(
        [gidx, jnp.zeros((pad,), jnp.int32)]).reshape(NSUB, NCHUNK, ECH)
    dst_t = jnp.concatenate(
        [dstv, jnp.full((pad,), N, jnp.int32)]).reshape(NSUB, NCHUNK, ECH)

    x8 = jnp.concatenate(
        [jnp.pad(x_low, ((0, NPAD - N), (0, 0))),
         jnp.pad(x_occ[:, :OUT - 1], ((0, NPAD - N), (0, 0)))], axis=1)

    cmask = (jnp.arange(OUT - 1)[None, :] <= jnp.arange(OUT - 1)[:, None])
    W0m = ot_W0 * cmask[:, None, :, None].astype(f32)
    wp1 = jnp.concatenate(
        [jnp.pad(bi_W0[None], ((0, 0), (0, 0), (0, OUT - 1), (0, 0))),
         jnp.pad(W0m, ((0, 0), (0, 0), (1, 0), (0, 0)))], axis=0)

    def blockdiag(w):
        B, Kk, Cb, Co = w.shape
        wt = jnp.moveaxis(w, 1, 0)
        out = jnp.zeros((Kk, B, Cb, B, Co), w.dtype)
        for b in range(B):
            out = out.at[:, b, :, b, :].set(wt[:, b])
        return out.reshape(Kk, B * Cb, B * Co)

    wp1 = blockdiag(wp1)
    w1 = blockdiag(jnp.concatenate([bi_W1[None], ot_W1], axis=0))
    w2 = blockdiag(jnp.concatenate([bi_W2[None], ot_W2], axis=0))
    w3 = blockdiag(jnp.concatenate([bi_W3[None], ot_W3], axis=0))
    wpr = blockdiag(pr_W)
    b0 = jnp.concatenate([bi_b0[None], ot_b0], axis=0).reshape(1, OUT * CH)
    b1 = jnp.concatenate([bi_b1[None], ot_b1], axis=0).reshape(1, OUT * CH)
    b2 = jnp.concatenate([bi_b2[None], ot_b2], axis=0).reshape(1, OUT * CH)
    b3 = jnp.concatenate([bi_b3[None], ot_b3], axis=0).reshape(1, OUT * CH)

    def agg(y):
        return _sc_agg(y, gidx_t, dst_t)

    yshape = (2, K, NPAD, FH)

    y = _call(_t_first, (x8, wp1), [_nspec(OUT), _fspec(wp1.shape)],
              yshape, _yspec())
    a1 = agg(y)
    y = _call(_t_lin, (a1, b0, w1),
              [_aspec(), _fspec(b0.shape), _fspec(w1.shape)],
              yshape, _yspec())
    a2 = agg(y)
    y = _call(_t_lin, (a2, b1, w2),
              [_aspec(), _fspec(b1.shape), _fspec(w2.shape)],
              yshape, _yspec())
    a3 = agg(y)
    y = _call(_t_res, (a1, a3, b0, b2, w3),
              [_aspec(), _aspec(), _fspec(b0.shape), _fspec(b2.shape),
               _fspec(w3.shape)],
              yshape, _yspec())
    a4 = agg(y)
    y = _call(_t_pr, (a4, b3, wpr),
              [_aspec(), _fspec(b3.shape), _fspec(wpr.shape)],
              yshape, _yspec())
    apr = agg(y)

    out = _call(_t_head, (apr, pr_b, ml_W1, ml_b1, ml_W2, ml_b2),
                [_aspec(), _fspec(pr_b.shape), _fspec(ml_W1.shape),
                 _fspec(ml_b1.shape), _fspec(ml_W2.shape),
                 _fspec(ml_b2.shape)],
                (NPAD, OUT), _nspec(OUT))
    return out[:N]


# speedup vs baseline: 1.2787x; 1.2787x over previous
import functools

import jax
import jax.numpy as jnp
from jax import lax
from jax.experimental import pallas as pl
from jax.experimental.pallas import tpu as pltpu
from jax.experimental.pallas import tpu_sc as plsc

N = 10000
E = 320000
K = 27
CH = 32
HID = 24
OUT = 8

NPAD = 10240
NB = 256
GRID = NPAD // NB
NSUB = 16
ECH = 128
NCHUNK = 160
IB = 32
NIB = NCHUNK // IB
EPAD = NSUB * NCHUNK * ECH
ZROWS = 16
RPS = NPAD // NSUB
FH = 128

f32 = jnp.float32


def _sc_body(y_hbm, gidx_hbm, dst_hbm, out_hbm,
             gidx_v, dst_v, zbuf, gb0, gb1, acc,
             sg0, sg1, ss0, ss1):
    c = lax.axis_index("c")
    s = lax.axis_index("s")
    gb = (gb0, gb1)
    sg = (sg0, sg1)
    ss = (ss0, ss1)

    def zrow(i, carry):
        for j in range(FH // 16):
            zbuf[i, pl.ds(j * 16, 16)] = jnp.zeros((16,), f32)
        return carry
    lax.fori_loop(0, ZROWS, zrow, 0)
    for r in range(RPS // ZROWS):
        pltpu.sync_copy(zbuf, acc.at[pl.ds(s * RPS + r * ZROWS, ZROWS)])
    plsc.subcore_barrier()

    yc = y_hbm.at[c]

    def blk(ib, carry):
        pltpu.sync_copy(gidx_hbm.at[s, pl.ds(ib * IB, IB)], gidx_v.at[0])
        pltpu.sync_copy(dst_hbm.at[s, pl.ds(ib * IB, IB)], dst_v.at[0])

        g_desc = [None, None]
        s_desc = [None, None]
        g_desc[0] = pltpu.async_copy(yc.at[gidx_v.at[0, 0]], gb[0], sg[0])
        for j in range(IB):
            p = j % 2
            q = 1 - p
            if j + 1 < IB:
                if j >= 1:
                    s_desc[q].wait()
                g_desc[q] = pltpu.async_copy(
                    yc.at[gidx_v.at[0, j + 1]], gb[q], sg[q])
            g_desc[p].wait()
            s_desc[p] = pltpu.async_copy(
                gb[p], acc.at[dst_v.at[0, j]], ss[p], add=True)
        s_desc[0].wait()
        s_desc[1].wait()
        return carry
    lax.fori_loop(0, NIB, blk, 0)
    plsc.subcore_barrier()

    pltpu.sync_copy(acc.at[pl.ds(s * RPS, RPS)],
                    out_hbm.at[c, pl.ds(s * RPS, RPS)])


def _sc_agg(y, gidx, dst):
    mesh = plsc.VectorSubcoreMesh(core_axis_name="c", subcore_axis_name="s")
    fn = pl.kernel(
        _sc_body,
        out_type=jax.ShapeDtypeStruct((2, NPAD, FH), f32),
        mesh=mesh,
        scratch_types=[
            pltpu.VMEM((1, IB, ECH), jnp.int32),
            pltpu.VMEM((1, IB, ECH), jnp.int32),
            pltpu.VMEM((ZROWS, FH), f32),
            pltpu.VMEM((ECH, FH), f32),
            pltpu.VMEM((ECH, FH), f32),
            pltpu.VMEM_SHARED((NPAD, FH), f32),
            pltpu.SemaphoreType.DMA,
            pltpu.SemaphoreType.DMA,
            pltpu.SemaphoreType.DMA,
            pltpu.SemaphoreType.DMA,
        ],
    )
    return fn(y.reshape(2, K * NPAD, FH), gidx, dst)


def _yspec():
    return pl.BlockSpec((2, K, NB, FH), lambda i: (0, 0, i, 0))


def _aspec():
    return pl.BlockSpec((2, NB, FH), lambda i: (0, i, 0))


def _nspec(w):
    return pl.BlockSpec((NB, w), lambda i: (i, 0))


def _fspec(shape):
    return pl.BlockSpec(shape, lambda i: (0,) * len(shape))


def _cat(a_ref):
    return jnp.concatenate([a_ref[0], a_ref[1]], axis=-1)


def _emit(h8, w_ref, y_ref):
    for k in range(K):
        row = jnp.dot(h8, w_ref[k], preferred_element_type=f32)
        y_ref[0, k] = row[:, :FH]
        y_ref[1, k] = row[:, FH:]


def _t_first(x_ref, w_ref, y_ref):
    x = x_ref[...]
    h8 = jnp.concatenate([x] * OUT, axis=1)
    _emit(h8, w_ref, y_ref)


def _t_lin(a_ref, b0_ref, w_ref, y_ref):
    a = _cat(a_ref)
    h8 = jnp.maximum(a + b0_ref[0], 0.0)
    _emit(h8, w_ref, y_ref)


def _t_res(a1_ref, a3_ref, b0_ref, b2_ref, w_ref, y_ref):
    h = jnp.maximum(_cat(a1_ref) + b0_ref[0], 0.0)
    r2 = _cat(a3_ref) + b2_ref[0]
    h8 = jnp.maximum(h + r2, 0.0)
    _emit(h8, w_ref, y_ref)


def _t_pr(a4_ref, b3_ref, w_ref, y_ref):
    a = _cat(a_ref=a4_ref)
    g = a + b3_ref[0]
    xg = g[:, :CH]
    parts = [xg] + [xg + g[:, oi * CH:(oi + 1) * CH] for oi in range(1, OUT)]
    h8 = jnp.concatenate(parts, axis=1)
    _emit(h8, w_ref, y_ref)


def _t_head(apr_ref, prb_ref, w1_ref, b1_ref, w2_ref, b2_ref, out_ref):
    a = _cat(apr_ref).reshape(NB, OUT, CH)
    cols = []
    for oi in range(OUT):
        t = a[:, oi] + prb_ref[oi]
        y1 = jnp.maximum(
            jnp.dot(t, w1_ref[oi], preferred_element_type=f32) + b1_ref[oi],
            0.0)
        z = jnp.dot(y1, w2_ref[oi], preferred_element_type=f32) + b2_ref[oi]
        cols.append(1.0 / (1.0 + jnp.exp(-z)))
    out_ref[...] = jnp.concatenate(cols, axis=1)


def _call(body, in_arrs, in_specs, out_shape, out_spec):
    return pl.pallas_call(
        body, grid=(GRID,), in_specs=in_specs,
        out_specs=out_spec,
        out_shape=jax.ShapeDtypeStruct(out_shape, f32),
    )(*in_arrs)


def kernel(x_low, x_occ, edge_index, edge_offset,
           bi_W0, bi_b0, bi_W1, bi_b1, bi_W2, bi_b2, bi_W3, bi_b3,
           pr_W, pr_b, ml_W1, ml_b1, ml_W2, ml_b2,
           ot_W0, ot_b0, ot_W1, ot_b1, ot_W2, ot_b2, ot_W3, ot_b3):
    src = edge_index[0]
    dstv = edge_index[1]
    off = edge_offset

    gidx = off * NPAD + src
    pad = EPAD - E
    gidx_t = jnp.concatenate(
        [gidx, jnp.zeros((pad,), jnp.int32)]).reshape(NSUB, NCHUNK, ECH)
    dst_t = jnp.concatenate(
        [dstv, jnp.full((pad,), N, jnp.int32)]).reshape(NSUB, NCHUNK, ECH)

    x8 = jnp.concatenate(
        [jnp.pad(x_low, ((0, NPAD - N), (0, 0))),
         jnp.pad(x_occ[:, :OUT - 1], ((0, NPAD - N), (0, 0)))], axis=1)

    cmask = (jnp.arange(OUT - 1)[None, :] <= jnp.arange(OUT - 1)[:, None])
    W0m = ot_W0 * cmask[:, None, :, None].astype(f32)
    wp1 = jnp.concatenate(
        [jnp.pad(bi_W0[None], ((0, 0), (0, 0), (0, OUT - 1), (0, 0))),
         jnp.pad(W0m, ((0, 0), (0, 0), (1, 0), (0, 0)))], axis=0)

    def blockdiag(w):
        B, Kk, Cb, Co = w.shape
        wt = jnp.moveaxis(w, 1, 0)
        out = jnp.zeros((Kk, B, Cb, B, Co), w.dtype)
        for b in range(B):
            out = out.at[:, b, :, b, :].set(wt[:, b])
        return out.reshape(Kk, B * Cb, B * Co)

    wp1 = blockdiag(wp1)
    w1 = blockdiag(jnp.concatenate([bi_W1[None], ot_W1], axis=0))
    w2 = blockdiag(jnp.concatenate([bi_W2[None], ot_W2], axis=0))
    w3 = blockdiag(jnp.concatenate([bi_W3[None], ot_W3], axis=0))
    wpr = blockdiag(pr_W)
    b0 = jnp.concatenate([bi_b0[None], ot_b0], axis=0).reshape(1, OUT * CH)
    b1 = jnp.concatenate([bi_b1[None], ot_b1], axis=0).reshape(1, OUT * CH)
    b2 = jnp.concatenate([bi_b2[None], ot_b2], axis=0).reshape(1, OUT * CH)
    b3 = jnp.concatenate([bi_b3[None], ot_b3], axis=0).reshape(1, OUT * CH)

    def agg(y):
        return _sc_agg(y, gidx_t, dst_t)

    yshape = (2, K, NPAD, FH)

    y = _call(_t_first, (x8, wp1), [_nspec(OUT), _fspec(wp1.shape)],
              yshape, _yspec())
    a1 = agg(y)
    y = _call(_t_lin, (a1, b0, w1),
              [_aspec(), _fspec(b0.shape), _fspec(w1.shape)],
              yshape, _yspec())
    a2 = agg(y)
    y = _call(_t_lin, (a2, b1, w2),
              [_aspec(), _fspec(b1.shape), _fspec(w2.shape)],
              yshape, _yspec())
    a3 = agg(y)
    y = _call(_t_res, (a1, a3, b0, b2, w3),
              [_aspec(), _aspec(), _fspec(b0.shape), _fspec(b2.shape),
               _fspec(w3.shape)],
              yshape, _yspec())
    a4 = agg(y)
    y = _call(_t_pr, (a4, b3, wpr),
              [_aspec(), _fspec(b3.shape), _fspec(wpr.shape)],
              yshape, _yspec())
    apr = agg(y)

    out = _call(_t_head, (apr, pr_b, ml_W1, ml_b1, ml_W2, ml_b2),
                [_aspec(), _fspec(pr_b.shape), _fspec(ml_W1.shape),
                 _fspec(ml_b1.shape), _fspec(ml_W2.shape),
                 _fspec(ml_b2.shape)],
                (NPAD, OUT), _nspec(OUT))
    return out[:N]
